# baseline (device time: 109935 ns/iter reference)
import jax
import jax.numpy as jnp
from jax import lax
from jax.experimental import pallas as pl
from jax.experimental.pallas import tpu as pltpu

N_DEV = 8
N_TILE = 2048
K_BLK = 128
NBUF = 3


def kernel(x, w_mat, scale_x, scale_w):
    m_total, k_per = x.shape
    k_total, n = w_mat.shape
    m_per = m_total // N_DEV
    n_blks = k_total // K_BLK
    blk_per_src = k_per // K_BLK
    n_tiles = n // N_TILE

    def body(x_ref, w_hbm, sx_ref, sw_ref, out_ref,
             comm_ref, stage_ref, wbuf_ref, send_sems, recv_sems, wsems):
        my = lax.axis_index("i")

        barrier_sem = pltpu.get_barrier_semaphore()
        for p in range(N_DEV):
            pl.semaphore_signal(
                barrier_sem, inc=1,
                device_id=(p,), device_id_type=pl.DeviceIdType.MESH,
            )
        pl.semaphore_wait(barrier_sem, N_DEV)

        def send_desc(d):
            return pltpu.make_async_remote_copy(
                src_ref=stage_ref.at[d],
                dst_ref=comm_ref.at[my],
                send_sem=send_sems.at[d],
                recv_sem=recv_sems.at[my],
                device_id=(d,),
                device_id_type=pl.DeviceIdType.MESH,
            )

        def recv_desc(src):
            return pltpu.make_async_remote_copy(
                src_ref=comm_ref.at[src],
                dst_ref=comm_ref.at[src],
                send_sem=send_sems.at[src],
                recv_sem=recv_sems.at[src],
                device_id=(src,),
                device_id_type=pl.DeviceIdType.MESH,
            )

        def wcopy(b):
            return pltpu.make_async_copy(
                w_hbm.at[pl.ds(b * K_BLK, K_BLK), :],
                wbuf_ref.at[b % NBUF],
                wsems.at[b % 4],
            )

        for d in range(N_DEV):
            stage_ref[d] = x_ref[pl.ds(d * m_per, m_per), :].astype(
                jnp.float8_e5m2
            )
        for d in range(N_DEV):
            send_desc(d).start()

        wcopy(0).start()
        wcopy(1).start()
        for b in range(n_blks):
            if b + 2 < n_blks:
                wcopy(b + 2).start()
            wcopy(b).wait()
            src = b // blk_per_src
            if b % blk_per_src == 0:
                recv_desc(src).wait_recv()
            a = comm_ref[src, :, pl.ds((b % blk_per_src) * K_BLK, K_BLK)]
            a = a.astype(jnp.bfloat16)
            for nt in range(n_tiles):
                part = jnp.dot(
                    a,
                    wbuf_ref[b % NBUF, :, pl.ds(nt * N_TILE, N_TILE)].astype(
                        jnp.bfloat16
                    ),
                    preferred_element_type=jnp.float32,
                )
                osl = (slice(None), pl.ds(nt * N_TILE, N_TILE))
                if b == 0:
                    out_ref[osl] = part
                else:
                    out_ref[osl] += part

        out_ref[...] *= sx_ref[0] * sw_ref[0]

        for d in range(N_DEV):
            send_desc(d).wait_send()

    return pl.pallas_call(
        body,
        out_shape=jax.ShapeDtypeStruct((m_per, n), jnp.float32),
        in_specs=[
            pl.BlockSpec(memory_space=pltpu.VMEM),
            pl.BlockSpec(memory_space=pltpu.MemorySpace.HBM),
            pl.BlockSpec(memory_space=pltpu.SMEM),
            pl.BlockSpec(memory_space=pltpu.SMEM),
        ],
        out_specs=pl.BlockSpec(memory_space=pltpu.VMEM),
        scratch_shapes=[
            pltpu.VMEM((N_DEV, m_per, k_per), jnp.float8_e5m2),
            pltpu.VMEM((N_DEV, m_per, k_per), jnp.float8_e5m2),
            pltpu.VMEM((NBUF, K_BLK, n), jnp.float32),
            pltpu.SemaphoreType.DMA((N_DEV,)),
            pltpu.SemaphoreType.DMA((N_DEV,)),
            pltpu.SemaphoreType.DMA((4,)),
        ],
        compiler_params=pltpu.CompilerParams(
            collective_id=0,
            vmem_limit_bytes=100 * 1024 * 1024,
        ),
    )(x, w_mat, scale_x, scale_w)


# device time: 90713 ns/iter; 1.2119x vs baseline; 1.2119x over previous
import jax
import jax.numpy as jnp
from jax import lax
from jax.experimental import pallas as pl
from jax.experimental.pallas import tpu as pltpu

N_DEV = 8
N_TILE = 2048
K_BLK = 128
NBUF = 3


def kernel(x, w_mat, scale_x, scale_w):
    m_total, k_per = x.shape
    k_total, n = w_mat.shape
    m_per = m_total // N_DEV
    n_blks = k_total // K_BLK
    blk_per_src = k_per // K_BLK
    n_tiles = n // N_TILE

    def body(x_ref, w_hbm, sx_ref, sw_ref, out_ref,
             comm_ref, stage_ref, wbuf_ref, send_sems, recv_sems, wsems):
        my = lax.axis_index("i")


        def send_desc(d):
            return pltpu.make_async_remote_copy(
                src_ref=stage_ref.at[d],
                dst_ref=comm_ref.at[my],
                send_sem=send_sems.at[d],
                recv_sem=recv_sems.at[my],
                device_id=(d,),
                device_id_type=pl.DeviceIdType.MESH,
            )

        def recv_desc(src):
            return pltpu.make_async_remote_copy(
                src_ref=comm_ref.at[src],
                dst_ref=comm_ref.at[src],
                send_sem=send_sems.at[src],
                recv_sem=recv_sems.at[src],
                device_id=(src,),
                device_id_type=pl.DeviceIdType.MESH,
            )

        def wcopy(b):
            return pltpu.make_async_copy(
                w_hbm.at[pl.ds(b * K_BLK, K_BLK), :],
                wbuf_ref.at[b % NBUF],
                wsems.at[b % 4],
            )

        for d in range(N_DEV):
            stage_ref[d] = x_ref[pl.ds(d * m_per, m_per), :].astype(
                jnp.float8_e5m2
            )
        ABLATE_COMM = True
        if not ABLATE_COMM:
            for d in range(N_DEV):
                send_desc(d).start()

        wcopy(0).start()
        wcopy(1).start()
        for b in range(n_blks):
            if b + 2 < n_blks:
                wcopy(b + 2).start()
            wcopy(b).wait()
            src = b // blk_per_src
            if not ABLATE_COMM and b % blk_per_src == 0:
                recv_desc(src).wait_recv()
            a = stage_ref[src, :, pl.ds((b % blk_per_src) * K_BLK, K_BLK)]
            a = a.astype(jnp.bfloat16)
            for nt in range(n_tiles):
                part = jnp.dot(
                    a,
                    wbuf_ref[b % NBUF, :, pl.ds(nt * N_TILE, N_TILE)].astype(
                        jnp.bfloat16
                    ),
                    preferred_element_type=jnp.float32,
                )
                osl = (slice(None), pl.ds(nt * N_TILE, N_TILE))
                if b == 0:
                    out_ref[osl] = part
                else:
                    out_ref[osl] += part

        out_ref[...] *= sx_ref[0] * sw_ref[0]

        if not ABLATE_COMM:
            for d in range(N_DEV):
                send_desc(d).wait_send()

    return pl.pallas_call(
        body,
        out_shape=jax.ShapeDtypeStruct((m_per, n), jnp.float32),
        in_specs=[
            pl.BlockSpec(memory_space=pltpu.VMEM),
            pl.BlockSpec(memory_space=pltpu.MemorySpace.HBM),
            pl.BlockSpec(memory_space=pltpu.SMEM),
            pl.BlockSpec(memory_space=pltpu.SMEM),
        ],
        out_specs=pl.BlockSpec(memory_space=pltpu.VMEM),
        scratch_shapes=[
            pltpu.VMEM((N_DEV, m_per, k_per), jnp.float8_e5m2),
            pltpu.VMEM((N_DEV, m_per, k_per), jnp.float8_e5m2),
            pltpu.VMEM((NBUF, K_BLK, n), jnp.float32),
            pltpu.SemaphoreType.DMA((N_DEV,)),
            pltpu.SemaphoreType.DMA((N_DEV,)),
            pltpu.SemaphoreType.DMA((4,)),
        ],
        compiler_params=pltpu.CompilerParams(
            vmem_limit_bytes=100 * 1024 * 1024,
        ),
    )(x, w_mat, scale_x, scale_w)


# device time: 84404 ns/iter; 1.3025x vs baseline; 1.0747x over previous
import jax
import jax.numpy as jnp
from jax import lax
from jax.experimental import pallas as pl
from jax.experimental.pallas import tpu as pltpu

N_DEV = 8
N_TILE = 2048
NBUF = 3


def kernel(x, w_mat, scale_x, scale_w):
    m_total, k_per = x.shape
    k_total, n = w_mat.shape
    m_per = m_total // N_DEV
    n_panels = n // N_TILE
    n_chunks = n_panels * N_DEV

    def body(x_ref, w_hbm, sx_ref, sw_ref, out_ref,
             comm_ref, stage_ref, a_ref, wf32_ref, w8_ref,
             send_sems, recv_sems, wsems):
        my = lax.axis_index("i")

        barrier_sem = pltpu.get_barrier_semaphore()
        for p in range(N_DEV):
            pl.semaphore_signal(
                barrier_sem, inc=1,
                device_id=(p,), device_id_type=pl.DeviceIdType.MESH,
            )
        pl.semaphore_wait(barrier_sem, N_DEV)

        def send_desc(d):
            return pltpu.make_async_remote_copy(
                src_ref=stage_ref.at[d],
                dst_ref=comm_ref.at[my],
                send_sem=send_sems.at[d],
                recv_sem=recv_sems.at[my],
                device_id=(d,),
                device_id_type=pl.DeviceIdType.MESH,
            )

        def recv_desc(src):
            return pltpu.make_async_remote_copy(
                src_ref=comm_ref.at[src],
                dst_ref=comm_ref.at[src],
                send_sem=send_sems.at[src],
                recv_sem=recv_sems.at[src],
                device_id=(src,),
                device_id_type=pl.DeviceIdType.MESH,
            )

        def wcopy(k):
            s, p = k % N_DEV, k // N_DEV
            return pltpu.make_async_copy(
                w_hbm.at[pl.ds(s * k_per, k_per), pl.ds(p * N_TILE, N_TILE)],
                wf32_ref.at[k % NBUF],
                wsems.at[k % 4],
            )

        for d in range(N_DEV):
            stage_ref[d] = x_ref[pl.ds(d * m_per, m_per), :].astype(
                jnp.float8_e5m2
            )
        for d in range(N_DEV):
            send_desc(d).start()

        scale = sx_ref[0] * sw_ref[0]

        wcopy(0).start()
        wcopy(1).start()
        for k in range(n_chunks):
            if k + 2 < n_chunks:
                wcopy(k + 2).start()
            wcopy(k).wait()
            s, p = k % N_DEV, k // N_DEV
            w8_ref[pl.ds(s * k_per, k_per), :] = wf32_ref[k % NBUF].astype(
                jnp.float8_e5m2
            )
            if s == N_DEV - 1:
                if p == 0:
                    for src in range(N_DEV):
                        recv_desc(src).wait_recv()
                        a_ref[:, pl.ds(src * k_per, k_per)] = comm_ref[src]
                part = jnp.dot(
                    a_ref[...], w8_ref[...],
                    preferred_element_type=jnp.float32,
                )
                out_ref[:, pl.ds(p * N_TILE, N_TILE)] = part * scale

        for d in range(N_DEV):
            send_desc(d).wait_send()

    return pl.pallas_call(
        body,
        out_shape=jax.ShapeDtypeStruct((m_per, n), jnp.float32),
        in_specs=[
            pl.BlockSpec(memory_space=pltpu.VMEM),
            pl.BlockSpec(memory_space=pltpu.MemorySpace.HBM),
            pl.BlockSpec(memory_space=pltpu.SMEM),
            pl.BlockSpec(memory_space=pltpu.SMEM),
        ],
        out_specs=pl.BlockSpec(memory_space=pltpu.VMEM),
        scratch_shapes=[
            pltpu.VMEM((N_DEV, m_per, k_per), jnp.float8_e5m2),
            pltpu.VMEM((N_DEV, m_per, k_per), jnp.float8_e5m2),
            pltpu.VMEM((m_per, k_total), jnp.float8_e5m2),
            pltpu.VMEM((NBUF, k_per, N_TILE), jnp.float32),
            pltpu.VMEM((k_total, N_TILE), jnp.float8_e5m2),
            pltpu.SemaphoreType.DMA((N_DEV,)),
            pltpu.SemaphoreType.DMA((N_DEV,)),
            pltpu.SemaphoreType.DMA((4,)),
        ],
        compiler_params=pltpu.CompilerParams(
            collective_id=0,
            vmem_limit_bytes=100 * 1024 * 1024,
        ),
    )(x, w_mat, scale_x, scale_w)
